# baseline (speedup 1.0000x reference)
import functools

import jax
import jax.numpy as jnp
from jax import lax
from jax.experimental import pallas as pl
from jax.experimental.pallas import tpu as pltpu
from jax.experimental.pallas import tpu_sc as plsc

NC = 2
NS = 16
NW = NC * NS
K = 128
D = 128
LANES = 16


def _mesh():
    return plsc.VectorSubcoreMesh(
        core_axis_name="c", subcore_axis_name="s",
        num_cores=NC, num_subcores=NS)


def _round_up(x, m):
    return (x + m - 1) // m * m


def _make_deg(nch, npad):
    rpt = npad // NS

    @functools.partial(
        pl.kernel,
        out_type=jax.ShapeDtypeStruct((NC, NS, rpt), jnp.float32),
        mesh=_mesh(),
        scratch_types=[
            pltpu.VMEM((nch, K), jnp.int32),
            pltpu.VMEM((K,), jnp.float32),
            pltpu.VMEM((rpt,), jnp.float32),
            pltpu.VMEM_SHARED((npad,), jnp.float32),
        ],
    )
    def deg_kernel(colr_hbm, out_hbm, idxc_v, ones_v, dbuf_v, dacc_sh):
        c = lax.axis_index("c")
        s = lax.axis_index("s")
        wid = c * NS + s
        zero16 = jnp.zeros((LANES,), jnp.float32)
        one16 = jnp.ones((LANES,), jnp.float32)

        def _z(i, _):
            dbuf_v[pl.ds(i * LANES, LANES)] = zero16
            return 0
        lax.fori_loop(0, rpt // LANES, _z, 0)

        def _o(i, _):
            ones_v[pl.ds(i * LANES, LANES)] = one16
            return 0
        lax.fori_loop(0, K // LANES, _o, 0)

        pltpu.sync_copy(dbuf_v, dacc_sh.at[pl.ds(s * rpt, rpt)])
        plsc.subcore_barrier()

        pltpu.sync_copy(colr_hbm.at[wid], idxc_v)

        def _chunk(j, _):
            pltpu.sync_copy(ones_v, dacc_sh.at[idxc_v.at[j]], add=True)
            return 0
        lax.fori_loop(0, nch, _chunk, 0)
        plsc.subcore_barrier()

        pltpu.sync_copy(dacc_sh.at[pl.ds(s * rpt, rpt)], dbuf_v)
        pltpu.sync_copy(dbuf_v, out_hbm.at[c, s])

    return deg_kernel


def _make_layer(nch, npad):
    rpt = npad // NS
    nblk = rpt // K
    nch2 = nch // 2
    assert rpt % K == 0 and nch % 4 == 0

    @functools.partial(
        pl.kernel,
        out_type=jax.ShapeDtypeStruct((NC, npad, D), jnp.float32),
        mesh=_mesh(),
        scratch_types=[
            pltpu.VMEM((nch2, K), jnp.int32),
            pltpu.VMEM((nch2, K), jnp.int32),
            pltpu.VMEM((K, D), jnp.float32),
            pltpu.VMEM((K, D), jnp.float32),
            pltpu.VMEM_SHARED((npad, D), jnp.float32),
            pltpu.SemaphoreType.DMA,
            pltpu.SemaphoreType.DMA,
            pltpu.SemaphoreType.DMA,
            pltpu.SemaphoreType.DMA,
        ],
    )
    def layer_kernel(z_hbm, rowr_hbm, colr_hbm, out_hbm,
                     idxr_v, idxc_v, rows0_v, rows1_v, acc_sh,
                     gsem0, gsem1, ssem0, ssem1):
        c = lax.axis_index("c")
        s = lax.axis_index("s")
        wid = c * NS + s
        zero16 = jnp.zeros((LANES,), jnp.float32)

        def _zrow(r, _):
            for j in range(D // LANES):
                rows0_v[r, pl.ds(j * LANES, LANES)] = zero16
            return 0
        lax.fori_loop(0, K, _zrow, 0)
        for b in range(nblk):
            pltpu.sync_copy(rows0_v, acc_sh.at[pl.ds(s * rpt + b * K, K)])
        plsc.subcore_barrier()

        for p in range(2):
            pltpu.sync_copy(rowr_hbm.at[wid * 2 + p], idxr_v)
            pltpu.sync_copy(colr_hbm.at[wid * 2 + p], idxc_v)

            pltpu.async_copy(z_hbm.at[idxr_v.at[0]], rows0_v, gsem0)
            pltpu.async_copy(z_hbm.at[idxr_v.at[1]], rows1_v, gsem1)

            def _pair(jj, _):
                j0 = 2 * jj
                pltpu.make_async_copy(
                    z_hbm.at[idxr_v.at[j0]], rows0_v, gsem0).wait()
                pltpu.async_copy(rows0_v, acc_sh.at[idxc_v.at[j0]], ssem0,
                                 add=True)
                pltpu.make_async_copy(
                    z_hbm.at[idxr_v.at[j0 + 1]], rows1_v, gsem1).wait()
                pltpu.async_copy(rows1_v, acc_sh.at[idxc_v.at[j0 + 1]],
                                 ssem1, add=True)

                @pl.when(j0 + 2 < nch2)
                def _():
                    pltpu.make_async_copy(
                        rows0_v, acc_sh.at[idxc_v.at[j0]], ssem0).wait()
                    pltpu.async_copy(
                        z_hbm.at[idxr_v.at[j0 + 2]], rows0_v, gsem0)
                    pltpu.make_async_copy(
                        rows1_v, acc_sh.at[idxc_v.at[j0 + 1]], ssem1).wait()
                    pltpu.async_copy(
                        z_hbm.at[idxr_v.at[j0 + 3]], rows1_v, gsem1)
                return 0
            lax.fori_loop(0, nch2 // 2, _pair, 0)

            pltpu.make_async_copy(
                rows0_v, acc_sh.at[idxc_v.at[nch2 - 2]], ssem0).wait()
            pltpu.make_async_copy(
                rows1_v, acc_sh.at[idxc_v.at[nch2 - 1]], ssem1).wait()
        plsc.subcore_barrier()

        for b in range(nblk):
            base = s * rpt + b * K
            pltpu.sync_copy(acc_sh.at[pl.ds(base, K)], rows0_v)
            pltpu.sync_copy(rows0_v, out_hbm.at[c, pl.ds(base, K)])

    return layer_kernel


def _d_of(degp):
    deg = degp[0] + degp[1]
    return jnp.where(deg > 0.0, lax.rsqrt(deg), 0.0)


def _tc_scale0_body(degp_ref, w_ref, z0_ref):
    d = _d_of(degp_ref[...])
    z0_ref[...] = d * w_ref[...]


def _tc_mid_body(degp_ref, sp_ref, w_ref, z1_ref, acc_ref):
    d = _d_of(degp_ref[...])
    ds_ = d * (sp_ref[0] + sp_ref[1])
    z1_ref[...] = d * ds_
    acc_ref[...] = w_ref[...] + ds_


def _tc_final_body(degp_ref, sp_ref, acc_ref, out_ref):
    d = _d_of(degp_ref[...])
    out_ref[...] = (acc_ref[...] + d * (sp_ref[0] + sp_ref[1])) * (1.0 / 3.0)


def _tc_call(body, degp3, arrays, n_out, npad):
    brc = 2048
    grid = (npad // brc,)
    degp_spec = pl.BlockSpec((NC, brc, 1), lambda i: (0, i, 0))
    mat_spec = pl.BlockSpec((brc, D), lambda i: (i, 0))
    part_spec = pl.BlockSpec((NC, brc, D), lambda i: (0, i, 0))
    in_specs = [degp_spec]
    for a in arrays:
        in_specs.append(part_spec if a.ndim == 3 else mat_spec)
    out_shape = tuple(
        jax.ShapeDtypeStruct((npad, D), jnp.float32) for _ in range(n_out))
    out_specs = tuple(mat_spec for _ in range(n_out))
    if n_out == 1:
        out_shape, out_specs = out_shape[0], out_specs[0]
    return pl.pallas_call(
        body, grid=grid, in_specs=in_specs,
        out_specs=out_specs, out_shape=out_shape,
    )(degp3, *arrays)


def kernel(edge_index, W):
    n, dm = W.shape
    e = edge_index.shape[1]
    assert dm == D
    npad = _round_up(n + 1, NS * K)
    nch = _round_up(-(-e // (NW * K)), 4)
    e_pad = NW * nch * K

    row = edge_index[0].astype(jnp.int32)
    col = edge_index[1].astype(jnp.int32)
    pad_n = e_pad - e
    spare = npad - n
    pad_idx = n + jnp.arange(pad_n, dtype=jnp.int32) % spare
    rowr = jnp.concatenate([row, pad_idx]).reshape(NW, nch, K)
    colr = jnp.concatenate([col, pad_idx]).reshape(NW, nch, K)
    w_pad = jnp.pad(W, ((0, npad - n), (0, 0)))

    degp = _make_deg(nch, npad)(colr)
    degp3 = degp.reshape(NC, npad, 1)

    rowr2 = rowr.reshape(NW * 2, nch // 2, K)
    colr2 = colr.reshape(NW * 2, nch // 2, K)

    layer = _make_layer(nch, npad)
    z0 = _tc_call(_tc_scale0_body, degp3, [w_pad], 1, npad)
    s1p = layer(z0, rowr2, colr2)
    z1, acc01 = _tc_call(_tc_mid_body, degp3, [s1p, w_pad], 2, npad)
    s2p = layer(z1, rowr2, colr2)
    outp = _tc_call(_tc_final_body, degp3, [s2p, acc01], 1, npad)

    emb = outp[:n]
    nu = n // 2
    return (emb[:nu], emb[nu:])


# speedup vs baseline: 1.0030x; 1.0030x over previous
import functools

import jax
import jax.numpy as jnp
from jax import lax
from jax.experimental import pallas as pl
from jax.experimental.pallas import tpu as pltpu
from jax.experimental.pallas import tpu_sc as plsc

NC = 2
NS = 16
NW = NC * NS
K = 128
D = 128
LANES = 16


def _mesh():
    return plsc.VectorSubcoreMesh(
        core_axis_name="c", subcore_axis_name="s",
        num_cores=NC, num_subcores=NS)


def _round_up(x, m):
    return (x + m - 1) // m * m


def _make_deg(nch, npad):
    rpt = npad // NS

    @functools.partial(
        pl.kernel,
        out_type=jax.ShapeDtypeStruct((NC, NS, rpt), jnp.float32),
        mesh=_mesh(),
        scratch_types=[
            pltpu.VMEM((nch, K), jnp.int32),
            pltpu.VMEM((K,), jnp.float32),
            pltpu.VMEM((rpt,), jnp.float32),
            pltpu.VMEM_SHARED((npad,), jnp.float32),
        ],
    )
    def deg_kernel(colr_hbm, out_hbm, idxc_v, ones_v, dbuf_v, dacc_sh):
        c = lax.axis_index("c")
        s = lax.axis_index("s")
        wid = c * NS + s
        zero16 = jnp.zeros((LANES,), jnp.float32)
        one16 = jnp.ones((LANES,), jnp.float32)

        def _z(i, _):
            dbuf_v[pl.ds(i * LANES, LANES)] = zero16
            return 0
        lax.fori_loop(0, rpt // LANES, _z, 0)

        def _o(i, _):
            ones_v[pl.ds(i * LANES, LANES)] = one16
            return 0
        lax.fori_loop(0, K // LANES, _o, 0)

        pltpu.sync_copy(dbuf_v, dacc_sh.at[pl.ds(s * rpt, rpt)])
        plsc.subcore_barrier()

        pltpu.sync_copy(colr_hbm.at[wid], idxc_v)

        def _chunk(j, _):
            pltpu.sync_copy(ones_v, dacc_sh.at[idxc_v.at[j]], add=True)
            return 0
        lax.fori_loop(0, nch, _chunk, 0)
        plsc.subcore_barrier()

        pltpu.sync_copy(dacc_sh.at[pl.ds(s * rpt, rpt)], dbuf_v)
        pltpu.sync_copy(dbuf_v, out_hbm.at[c, s])

    return deg_kernel


def _make_layer(nch, npad):
    rpt = npad // NS
    nblk = rpt // K
    nch2 = nch // 2
    assert rpt % K == 0 and nch % 4 == 0

    @functools.partial(
        pl.kernel,
        out_type=jax.ShapeDtypeStruct((NC, npad, D), jnp.float32),
        mesh=_mesh(),
        scratch_types=[
            pltpu.VMEM((nch2, K), jnp.int32),
            pltpu.VMEM((nch2, K), jnp.int32),
            pltpu.VMEM((K, D), jnp.float32),
            pltpu.VMEM((K, D), jnp.float32),
            pltpu.VMEM_SHARED((npad, D), jnp.float32),
            pltpu.SemaphoreType.DMA,
            pltpu.SemaphoreType.DMA,
            pltpu.SemaphoreType.DMA,
            pltpu.SemaphoreType.DMA,
        ],
    )
    def layer_kernel(z_hbm, rowr_hbm, colr_hbm, out_hbm,
                     idxr_v, idxc_v, rows0_v, rows1_v, acc_sh,
                     gsem0, gsem1, ssem0, ssem1):
        c = lax.axis_index("c")
        s = lax.axis_index("s")
        wid = c * NS + s
        zero16 = jnp.zeros((LANES,), jnp.float32)

        def _zrow(r, _):
            for j in range(D // LANES):
                rows0_v[r, pl.ds(j * LANES, LANES)] = zero16
            return 0
        lax.fori_loop(0, K, _zrow, 0)
        for b in range(nblk):
            pltpu.async_copy(
                rows0_v, acc_sh.at[pl.ds(s * rpt + b * K, K)], gsem0)
        for b in range(nblk):
            pltpu.make_async_copy(
                rows0_v, acc_sh.at[pl.ds(s * rpt + b * K, K)], gsem0).wait()
        plsc.subcore_barrier()

        for p in range(2):
            pltpu.sync_copy(rowr_hbm.at[wid * 2 + p], idxr_v)
            pltpu.sync_copy(colr_hbm.at[wid * 2 + p], idxc_v)

            pltpu.async_copy(z_hbm.at[idxr_v.at[0]], rows0_v, gsem0)
            pltpu.async_copy(z_hbm.at[idxr_v.at[1]], rows1_v, gsem1)

            def _pair(jj, _):
                j0 = 2 * jj
                pltpu.make_async_copy(
                    z_hbm.at[idxr_v.at[j0]], rows0_v, gsem0).wait()
                pltpu.async_copy(rows0_v, acc_sh.at[idxc_v.at[j0]], ssem0,
                                 add=True)
                pltpu.make_async_copy(
                    z_hbm.at[idxr_v.at[j0 + 1]], rows1_v, gsem1).wait()
                pltpu.async_copy(rows1_v, acc_sh.at[idxc_v.at[j0 + 1]],
                                 ssem1, add=True)

                @pl.when(j0 + 2 < nch2)
                def _():
                    pltpu.make_async_copy(
                        rows0_v, acc_sh.at[idxc_v.at[j0]], ssem0).wait()
                    pltpu.async_copy(
                        z_hbm.at[idxr_v.at[j0 + 2]], rows0_v, gsem0)
                    pltpu.make_async_copy(
                        rows1_v, acc_sh.at[idxc_v.at[j0 + 1]], ssem1).wait()
                    pltpu.async_copy(
                        z_hbm.at[idxr_v.at[j0 + 3]], rows1_v, gsem1)
                return 0
            lax.fori_loop(0, nch2 // 2, _pair, 0)

            pltpu.make_async_copy(
                rows0_v, acc_sh.at[idxc_v.at[nch2 - 2]], ssem0).wait()
            pltpu.make_async_copy(
                rows1_v, acc_sh.at[idxc_v.at[nch2 - 1]], ssem1).wait()
        plsc.subcore_barrier()

        for b in range(nblk):
            base = s * rpt + b * K
            pltpu.async_copy(
                acc_sh.at[pl.ds(base, K)], out_hbm.at[c, pl.ds(base, K)],
                ssem0)
        for b in range(nblk):
            base = s * rpt + b * K
            pltpu.make_async_copy(
                acc_sh.at[pl.ds(base, K)], out_hbm.at[c, pl.ds(base, K)],
                ssem0).wait()

    return layer_kernel


def _d_of(degp):
    deg = degp[0] + degp[1]
    return jnp.where(deg > 0.0, lax.rsqrt(deg), 0.0)


def _tc_scale0_body(degp_ref, w_ref, z0_ref):
    d = _d_of(degp_ref[...])
    z0_ref[...] = d * w_ref[...]


def _tc_mid_body(degp_ref, sp_ref, w_ref, z1_ref, acc_ref):
    d = _d_of(degp_ref[...])
    ds_ = d * (sp_ref[0] + sp_ref[1])
    z1_ref[...] = d * ds_
    acc_ref[...] = w_ref[...] + ds_


def _tc_final_body(degp_ref, sp_ref, acc_ref, out_ref):
    d = _d_of(degp_ref[...])
    out_ref[...] = (acc_ref[...] + d * (sp_ref[0] + sp_ref[1])) * (1.0 / 3.0)


def _tc_call(body, degp3, arrays, n_out, npad):
    brc = 2048
    grid = (npad // brc,)
    degp_spec = pl.BlockSpec((NC, brc, 1), lambda i: (0, i, 0))
    mat_spec = pl.BlockSpec((brc, D), lambda i: (i, 0))
    part_spec = pl.BlockSpec((NC, brc, D), lambda i: (0, i, 0))
    in_specs = [degp_spec]
    for a in arrays:
        in_specs.append(part_spec if a.ndim == 3 else mat_spec)
    out_shape = tuple(
        jax.ShapeDtypeStruct((npad, D), jnp.float32) for _ in range(n_out))
    out_specs = tuple(mat_spec for _ in range(n_out))
    if n_out == 1:
        out_shape, out_specs = out_shape[0], out_specs[0]
    return pl.pallas_call(
        body, grid=grid, in_specs=in_specs,
        out_specs=out_specs, out_shape=out_shape,
    )(degp3, *arrays)


def kernel(edge_index, W):
    n, dm = W.shape
    e = edge_index.shape[1]
    assert dm == D
    npad = _round_up(n + 1, NS * K)
    nch = _round_up(-(-e // (NW * K)), 4)
    e_pad = NW * nch * K

    row = edge_index[0].astype(jnp.int32)
    col = edge_index[1].astype(jnp.int32)
    pad_n = e_pad - e
    spare = npad - n
    pad_idx = n + jnp.arange(pad_n, dtype=jnp.int32) % spare
    rowr = jnp.concatenate([row, pad_idx]).reshape(NW, nch, K)
    colr = jnp.concatenate([col, pad_idx]).reshape(NW, nch, K)
    w_pad = jnp.pad(W, ((0, npad - n), (0, 0)))

    degp = _make_deg(nch, npad)(colr)
    degp3 = degp.reshape(NC, npad, 1)

    rowr2 = rowr.reshape(NW * 2, nch // 2, K)
    colr2 = colr.reshape(NW * 2, nch // 2, K)

    layer = _make_layer(nch, npad)
    z0 = _tc_call(_tc_scale0_body, degp3, [w_pad], 1, npad)
    s1p = layer(z0, rowr2, colr2)
    z1, acc01 = _tc_call(_tc_mid_body, degp3, [s1p, w_pad], 2, npad)
    s2p = layer(z1, rowr2, colr2)
    outp = _tc_call(_tc_final_body, degp3, [s2p, acc01], 1, npad)

    emb = outp[:n]
    nu = n // 2
    return (emb[:nu], emb[nu:])


# speedup vs baseline: 1.1914x; 1.1877x over previous
import functools

import jax
import jax.numpy as jnp
from jax import lax
from jax.experimental import pallas as pl
from jax.experimental.pallas import tpu as pltpu
from jax.experimental.pallas import tpu_sc as plsc

NC = 2
NS = 16
NW = NC * NS
K = 128
D = 128
LANES = 16


def _mesh():
    return plsc.VectorSubcoreMesh(
        core_axis_name="c", subcore_axis_name="s",
        num_cores=NC, num_subcores=NS)


def _round_up(x, m):
    return (x + m - 1) // m * m


def _make_deg(nch, npad):
    rpt = npad // NS

    @functools.partial(
        pl.kernel,
        out_type=jax.ShapeDtypeStruct((NC, NS, rpt), jnp.float32),
        mesh=_mesh(),
        scratch_types=[
            pltpu.VMEM((nch, K), jnp.int32),
            pltpu.VMEM((K,), jnp.float32),
            pltpu.VMEM((rpt,), jnp.float32),
            pltpu.VMEM_SHARED((npad,), jnp.float32),
        ],
    )
    def deg_kernel(colr_hbm, out_hbm, idxc_v, ones_v, dbuf_v, dacc_sh):
        c = lax.axis_index("c")
        s = lax.axis_index("s")
        wid = c * NS + s
        zero16 = jnp.zeros((LANES,), jnp.float32)
        one16 = jnp.ones((LANES,), jnp.float32)

        def _z(i, _):
            dbuf_v[pl.ds(i * LANES, LANES)] = zero16
            return 0
        lax.fori_loop(0, rpt // LANES, _z, 0)

        def _o(i, _):
            ones_v[pl.ds(i * LANES, LANES)] = one16
            return 0
        lax.fori_loop(0, K // LANES, _o, 0)

        pltpu.sync_copy(dbuf_v, dacc_sh.at[pl.ds(s * rpt, rpt)])
        plsc.subcore_barrier()

        pltpu.sync_copy(colr_hbm.at[wid], idxc_v)

        def _chunk(j, _):
            pltpu.sync_copy(ones_v, dacc_sh.at[idxc_v.at[j]], add=True)
            return 0
        lax.fori_loop(0, nch, _chunk, 0)
        plsc.subcore_barrier()

        pltpu.sync_copy(dacc_sh.at[pl.ds(s * rpt, rpt)], dbuf_v)
        pltpu.sync_copy(dbuf_v, out_hbm.at[c, s])

    return deg_kernel


CK = 64
NB = 4


def _make_layer(nch, npad):
    rpt = npad // NS
    nblk = rpt // K
    nch2 = nch // 2
    nck = nch2 * 2
    assert rpt % K == 0 and nch % 4 == 0 and nck % NB == 0

    @functools.partial(
        pl.kernel,
        out_type=jax.ShapeDtypeStruct((NC, npad, D), jnp.float32),
        mesh=_mesh(),
        scratch_types=[
            pltpu.VMEM((nch2, K), jnp.int32),
            pltpu.VMEM((nch2, K), jnp.int32),
            pltpu.VMEM((CK, D), jnp.float32),
            pltpu.VMEM((CK, D), jnp.float32),
            pltpu.VMEM((CK, D), jnp.float32),
            pltpu.VMEM((CK, D), jnp.float32),
            pltpu.VMEM_SHARED((npad, D), jnp.float32),
            pltpu.SemaphoreType.DMA,
            pltpu.SemaphoreType.DMA,
            pltpu.SemaphoreType.DMA,
            pltpu.SemaphoreType.DMA,
            pltpu.SemaphoreType.DMA,
            pltpu.SemaphoreType.DMA,
            pltpu.SemaphoreType.DMA,
            pltpu.SemaphoreType.DMA,
        ],
    )
    def layer_kernel(z_hbm, rowr_hbm, colr_hbm, out_hbm,
                     idxr_v, idxc_v, r0, r1, r2, r3, acc_sh,
                     g0, g1, g2, g3, s0, s1, s2, s3):
        c = lax.axis_index("c")
        s = lax.axis_index("s")
        wid = c * NS + s
        zero16 = jnp.zeros((LANES,), jnp.float32)
        rows = [r0, r1, r2, r3]
        gsems = [g0, g1, g2, g3]
        ssems = [s0, s1, s2, s3]

        def _ridx(jj, b):
            return 2 * jj + (b // 2), (b % 2) * CK

        def _zrow(r, _):
            for j in range(D // LANES):
                r0[r, pl.ds(j * LANES, LANES)] = zero16
            return 0
        lax.fori_loop(0, CK, _zrow, 0)
        for b in range(rpt // CK):
            pltpu.async_copy(
                r0, acc_sh.at[pl.ds(s * rpt + b * CK, CK)], g0)
        for b in range(rpt // CK):
            pltpu.make_async_copy(
                r0, acc_sh.at[pl.ds(s * rpt + b * CK, CK)], g0).wait()
        plsc.subcore_barrier()

        for p in range(2):
            pltpu.sync_copy(rowr_hbm.at[wid * 2 + p], idxr_v)
            pltpu.sync_copy(colr_hbm.at[wid * 2 + p], idxc_v)

            for b in range(NB):
                r, h = _ridx(0, b)
                pltpu.async_copy(
                    z_hbm.at[idxr_v.at[r, pl.ds(h, CK)]], rows[b], gsems[b])

            def _ring(jj, _):
                for b in range(NB):
                    r, h = _ridx(jj, b)
                    pltpu.make_async_copy(
                        z_hbm.at[idxr_v.at[r, pl.ds(h, CK)]], rows[b],
                        gsems[b]).wait()
                    pltpu.async_copy(
                        rows[b], acc_sh.at[idxc_v.at[r, pl.ds(h, CK)]],
                        ssems[b], add=True)
                for b in range(NB):
                    r, h = _ridx(jj, b)
                    pltpu.make_async_copy(
                        rows[b], acc_sh.at[idxc_v.at[r, pl.ds(h, CK)]],
                        ssems[b]).wait()
                    pltpu.async_copy(
                        z_hbm.at[idxr_v.at[r + 2, pl.ds(h, CK)]], rows[b],
                        gsems[b])
                return 0
            lax.fori_loop(0, nck // NB - 1, _ring, 0)

            for b in range(NB):
                r, h = _ridx(nck // NB - 1, b)
                pltpu.make_async_copy(
                    z_hbm.at[idxr_v.at[r, pl.ds(h, CK)]], rows[b],
                    gsems[b]).wait()
                pltpu.async_copy(
                    rows[b], acc_sh.at[idxc_v.at[r, pl.ds(h, CK)]],
                    ssems[b], add=True)
            for b in range(NB):
                r, h = _ridx(nck // NB - 1, b)
                pltpu.make_async_copy(
                    rows[b], acc_sh.at[idxc_v.at[r, pl.ds(h, CK)]],
                    ssems[b]).wait()
        plsc.subcore_barrier()

        for b in range(nblk):
            base = s * rpt + b * K
            pltpu.async_copy(
                acc_sh.at[pl.ds(base, K)], out_hbm.at[c, pl.ds(base, K)],
                s0)
        for b in range(nblk):
            base = s * rpt + b * K
            pltpu.make_async_copy(
                acc_sh.at[pl.ds(base, K)], out_hbm.at[c, pl.ds(base, K)],
                s0).wait()

    return layer_kernel


def _d_of(degp):
    deg = degp[0] + degp[1]
    return jnp.where(deg > 0.0, lax.rsqrt(deg), 0.0)


def _tc_scale0_body(degp_ref, w_ref, z0_ref):
    d = _d_of(degp_ref[...])
    z0_ref[...] = d * w_ref[...]


def _tc_mid_body(degp_ref, sp_ref, w_ref, z1_ref, acc_ref):
    d = _d_of(degp_ref[...])
    ds_ = d * (sp_ref[0] + sp_ref[1])
    z1_ref[...] = d * ds_
    acc_ref[...] = w_ref[...] + ds_


def _tc_final_body(degp_ref, sp_ref, acc_ref, out_ref):
    d = _d_of(degp_ref[...])
    out_ref[...] = (acc_ref[...] + d * (sp_ref[0] + sp_ref[1])) * (1.0 / 3.0)


def _tc_call(body, degp3, arrays, n_out, npad):
    brc = 2048
    grid = (npad // brc,)
    degp_spec = pl.BlockSpec((NC, brc, 1), lambda i: (0, i, 0))
    mat_spec = pl.BlockSpec((brc, D), lambda i: (i, 0))
    part_spec = pl.BlockSpec((NC, brc, D), lambda i: (0, i, 0))
    in_specs = [degp_spec]
    for a in arrays:
        in_specs.append(part_spec if a.ndim == 3 else mat_spec)
    out_shape = tuple(
        jax.ShapeDtypeStruct((npad, D), jnp.float32) for _ in range(n_out))
    out_specs = tuple(mat_spec for _ in range(n_out))
    if n_out == 1:
        out_shape, out_specs = out_shape[0], out_specs[0]
    return pl.pallas_call(
        body, grid=grid, in_specs=in_specs,
        out_specs=out_specs, out_shape=out_shape,
    )(degp3, *arrays)


def kernel(edge_index, W):
    n, dm = W.shape
    e = edge_index.shape[1]
    assert dm == D
    npad = _round_up(n + 1, NS * K)
    nch = _round_up(-(-e // (NW * K)), 4)
    e_pad = NW * nch * K

    row = edge_index[0].astype(jnp.int32)
    col = edge_index[1].astype(jnp.int32)
    pad_n = e_pad - e
    spare = npad - n
    pad_idx = n + jnp.arange(pad_n, dtype=jnp.int32) % spare
    rowr = jnp.concatenate([row, pad_idx]).reshape(NW, nch, K)
    colr = jnp.concatenate([col, pad_idx]).reshape(NW, nch, K)
    w_pad = jnp.pad(W, ((0, npad - n), (0, 0)))

    degp = _make_deg(nch, npad)(colr)
    degp3 = degp.reshape(NC, npad, 1)

    rowr2 = rowr.reshape(NW * 2, nch // 2, K)
    colr2 = colr.reshape(NW * 2, nch // 2, K)

    layer = _make_layer(nch, npad)
    z0 = _tc_call(_tc_scale0_body, degp3, [w_pad], 1, npad)
    s1p = layer(z0, rowr2, colr2)
    z1, acc01 = _tc_call(_tc_mid_body, degp3, [s1p, w_pad], 2, npad)
    s2p = layer(z1, rowr2, colr2)
    outp = _tc_call(_tc_final_body, degp3, [s2p, acc01], 1, npad)

    emb = outp[:n]
    nu = n // 2
    return (emb[:nu], emb[nu:])


# speedup vs baseline: 1.2125x; 1.0178x over previous
import functools

import jax
import jax.numpy as jnp
from jax import lax
from jax.experimental import pallas as pl
from jax.experimental.pallas import tpu as pltpu
from jax.experimental.pallas import tpu_sc as plsc

NC = 2
NS = 16
NW = NC * NS
K = 128
D = 128
LANES = 16


def _mesh():
    return plsc.VectorSubcoreMesh(
        core_axis_name="c", subcore_axis_name="s",
        num_cores=NC, num_subcores=NS)


def _round_up(x, m):
    return (x + m - 1) // m * m


def _make_deg(nch, npad):
    rpt = npad // NS

    @functools.partial(
        pl.kernel,
        out_type=jax.ShapeDtypeStruct((NC, NS, rpt), jnp.float32),
        mesh=_mesh(),
        scratch_types=[
            pltpu.VMEM((nch, K), jnp.int32),
            pltpu.VMEM((K,), jnp.float32),
            pltpu.VMEM((rpt,), jnp.float32),
            pltpu.VMEM_SHARED((npad,), jnp.float32),
        ],
    )
    def deg_kernel(colr_hbm, out_hbm, idxc_v, ones_v, dbuf_v, dacc_sh):
        c = lax.axis_index("c")
        s = lax.axis_index("s")
        wid = c * NS + s
        zero16 = jnp.zeros((LANES,), jnp.float32)
        one16 = jnp.ones((LANES,), jnp.float32)

        def _z(i, _):
            dbuf_v[pl.ds(i * LANES, LANES)] = zero16
            return 0
        lax.fori_loop(0, rpt // LANES, _z, 0)

        def _o(i, _):
            ones_v[pl.ds(i * LANES, LANES)] = one16
            return 0
        lax.fori_loop(0, K // LANES, _o, 0)

        pltpu.sync_copy(dbuf_v, dacc_sh.at[pl.ds(s * rpt, rpt)])
        plsc.subcore_barrier()

        pltpu.sync_copy(colr_hbm.at[wid], idxc_v)

        def _chunk(j, _):
            pltpu.sync_copy(ones_v, dacc_sh.at[idxc_v.at[j]], add=True)
            return 0
        lax.fori_loop(0, nch, _chunk, 0)
        plsc.subcore_barrier()

        pltpu.sync_copy(dacc_sh.at[pl.ds(s * rpt, rpt)], dbuf_v)
        pltpu.sync_copy(dbuf_v, out_hbm.at[c, s])

    return deg_kernel


CK = 32
NB = 8
PER = K // CK


def _make_layer(nch, npad):
    rpt = npad // NS
    nblk = rpt // K
    nch2 = nch // 2
    nck = nch2 * PER
    assert rpt % K == 0 and nch % 4 == 0 and nck % NB == 0

    @functools.partial(
        pl.kernel,
        out_type=jax.ShapeDtypeStruct((NC, npad, D), jnp.float32),
        mesh=_mesh(),
        scratch_types=[
            pltpu.VMEM((nch2, K), jnp.int32),
            pltpu.VMEM((nch2, K), jnp.int32),
            *[pltpu.VMEM((CK, D), jnp.float32) for _ in range(NB)],
            pltpu.VMEM_SHARED((npad, D), jnp.float32),
            *[pltpu.SemaphoreType.DMA for _ in range(2 * NB)],
        ],
    )
    def layer_kernel(z_hbm, rowr_hbm, colr_hbm, out_hbm,
                     idxr_v, idxc_v, *bufs):
        rows = list(bufs[:NB])
        acc_sh = bufs[NB]
        gsems = list(bufs[NB + 1:2 * NB + 1])
        ssems = list(bufs[2 * NB + 1:3 * NB + 1])
        r0, s0 = rows[0], ssems[0]
        g0 = gsems[0]
        c = lax.axis_index("c")
        s = lax.axis_index("s")
        wid = c * NS + s
        zero16 = jnp.zeros((LANES,), jnp.float32)

        def _ridx(jj, b):
            return (NB // PER) * jj + (b // PER), (b % PER) * CK

        def _zrow(r, _):
            for j in range(D // LANES):
                r0[r, pl.ds(j * LANES, LANES)] = zero16
            return 0
        lax.fori_loop(0, CK, _zrow, 0)
        for b in range(rpt // CK):
            pltpu.async_copy(
                r0, acc_sh.at[pl.ds(s * rpt + b * CK, CK)], g0)
        for b in range(rpt // CK):
            pltpu.make_async_copy(
                r0, acc_sh.at[pl.ds(s * rpt + b * CK, CK)], g0).wait()
        plsc.subcore_barrier()

        for p in range(2):
            pltpu.sync_copy(rowr_hbm.at[wid * 2 + p], idxr_v)
            pltpu.sync_copy(colr_hbm.at[wid * 2 + p], idxc_v)

            for b in range(NB):
                r, h = _ridx(0, b)
                pltpu.async_copy(
                    z_hbm.at[idxr_v.at[r, pl.ds(h, CK)]], rows[b], gsems[b])

            def _ring(jj, _):
                for b in range(NB):
                    r, h = _ridx(jj, b)
                    pltpu.make_async_copy(
                        z_hbm.at[idxr_v.at[r, pl.ds(h, CK)]], rows[b],
                        gsems[b]).wait()
                    pltpu.async_copy(
                        rows[b], acc_sh.at[idxc_v.at[r, pl.ds(h, CK)]],
                        ssems[b], add=True)
                for b in range(NB):
                    r, h = _ridx(jj, b)
                    pltpu.make_async_copy(
                        rows[b], acc_sh.at[idxc_v.at[r, pl.ds(h, CK)]],
                        ssems[b]).wait()
                    pltpu.async_copy(
                        z_hbm.at[idxr_v.at[r + 2, pl.ds(h, CK)]], rows[b],
                        gsems[b])
                return 0
            lax.fori_loop(0, nck // NB - 1, _ring, 0)

            for b in range(NB):
                r, h = _ridx(nck // NB - 1, b)
                pltpu.make_async_copy(
                    z_hbm.at[idxr_v.at[r, pl.ds(h, CK)]], rows[b],
                    gsems[b]).wait()
                pltpu.async_copy(
                    rows[b], acc_sh.at[idxc_v.at[r, pl.ds(h, CK)]],
                    ssems[b], add=True)
            for b in range(NB):
                r, h = _ridx(nck // NB - 1, b)
                pltpu.make_async_copy(
                    rows[b], acc_sh.at[idxc_v.at[r, pl.ds(h, CK)]],
                    ssems[b]).wait()
        plsc.subcore_barrier()

        for b in range(nblk):
            base = s * rpt + b * K
            pltpu.async_copy(
                acc_sh.at[pl.ds(base, K)], out_hbm.at[c, pl.ds(base, K)],
                s0)
        for b in range(nblk):
            base = s * rpt + b * K
            pltpu.make_async_copy(
                acc_sh.at[pl.ds(base, K)], out_hbm.at[c, pl.ds(base, K)],
                s0).wait()

    return layer_kernel


def _d_of(degp):
    deg = degp[0] + degp[1]
    return jnp.where(deg > 0.0, lax.rsqrt(deg), 0.0)


def _tc_scale0_body(degp_ref, w_ref, z0_ref):
    d = _d_of(degp_ref[...])
    z0_ref[...] = d * w_ref[...]


def _tc_mid_body(degp_ref, sp_ref, w_ref, z1_ref, acc_ref):
    d = _d_of(degp_ref[...])
    ds_ = d * (sp_ref[0] + sp_ref[1])
    z1_ref[...] = d * ds_
    acc_ref[...] = w_ref[...] + ds_


def _tc_final_body(degp_ref, sp_ref, acc_ref, out_ref):
    d = _d_of(degp_ref[...])
    out_ref[...] = (acc_ref[...] + d * (sp_ref[0] + sp_ref[1])) * (1.0 / 3.0)


def _tc_call(body, degp3, arrays, n_out, npad):
    brc = 2048
    grid = (npad // brc,)
    degp_spec = pl.BlockSpec((NC, brc, 1), lambda i: (0, i, 0))
    mat_spec = pl.BlockSpec((brc, D), lambda i: (i, 0))
    part_spec = pl.BlockSpec((NC, brc, D), lambda i: (0, i, 0))
    in_specs = [degp_spec]
    for a in arrays:
        in_specs.append(part_spec if a.ndim == 3 else mat_spec)
    out_shape = tuple(
        jax.ShapeDtypeStruct((npad, D), jnp.float32) for _ in range(n_out))
    out_specs = tuple(mat_spec for _ in range(n_out))
    if n_out == 1:
        out_shape, out_specs = out_shape[0], out_specs[0]
    return pl.pallas_call(
        body, grid=grid, in_specs=in_specs,
        out_specs=out_specs, out_shape=out_shape,
    )(degp3, *arrays)


def kernel(edge_index, W):
    n, dm = W.shape
    e = edge_index.shape[1]
    assert dm == D
    npad = _round_up(n + 1, NS * K)
    nch = _round_up(-(-e // (NW * K)), 4)
    e_pad = NW * nch * K

    row = edge_index[0].astype(jnp.int32)
    col = edge_index[1].astype(jnp.int32)
    pad_n = e_pad - e
    spare = npad - n
    pad_idx = n + jnp.arange(pad_n, dtype=jnp.int32) % spare
    rowr = jnp.concatenate([row, pad_idx]).reshape(NW, nch, K)
    colr = jnp.concatenate([col, pad_idx]).reshape(NW, nch, K)
    w_pad = jnp.pad(W, ((0, npad - n), (0, 0)))

    degp = _make_deg(nch, npad)(colr)
    degp3 = degp.reshape(NC, npad, 1)

    rowr2 = rowr.reshape(NW * 2, nch // 2, K)
    colr2 = colr.reshape(NW * 2, nch // 2, K)

    layer = _make_layer(nch, npad)
    z0 = _tc_call(_tc_scale0_body, degp3, [w_pad], 1, npad)
    s1p = layer(z0, rowr2, colr2)
    z1, acc01 = _tc_call(_tc_mid_body, degp3, [s1p, w_pad], 2, npad)
    s2p = layer(z1, rowr2, colr2)
    outp = _tc_call(_tc_final_body, degp3, [s2p, acc01], 1, npad)

    emb = outp[:n]
    nu = n // 2
    return (emb[:nu], emb[nu:])
